# baseline (speedup 1.0000x reference)
import functools

import jax
import jax.numpy as jnp
from jax import lax
from jax.experimental import pallas as pl
from jax.experimental.pallas import tpu as pltpu
from jax.experimental.pallas import tpu_sc as plsc

NUM_EMB = 1_000_000
EMB_DIM = 32
N_I = 16384
N_J = 50
NUM_WORKERS = 32
N_BLOCKS = N_J * (N_I // 128)
BLOCKS_PER_W = N_BLOCKS // NUM_WORKERS

_mesh = plsc.VectorSubcoreMesh(core_axis_name="c", subcore_axis_name="s")

WC = 512
CHUNKS_PW = 61
PW_COLS = WC * CHUNKS_PW
EXTRA_C0 = PW_COLS * NUM_WORKERS
TAIL_C0 = EXTRA_C0 + WC
TAIL = NUM_EMB - TAIL_C0


@functools.partial(
    pl.kernel,
    mesh=_mesh,
    out_type=jax.ShapeDtypeStruct((NUM_EMB // 4, 128), jnp.float32),
    scratch_types=[
        pltpu.VMEM((2, EMB_DIM, WC), jnp.float32),
        pltpu.VMEM((2, WC // 4, 128), jnp.float32),
    ] + [pltpu.SemaphoreType.DMA] * 4,
    compiler_params=pltpu.CompilerParams(needs_layout_passes=False),
)
def _sc_repack(tab_ref, tail_ref, rp_ref, slab_v, out_v, *sems):
    in_sems, wr_sems = sems[:2], sems[2:]
    wid = lax.axis_index("s") * 2 + lax.axis_index("c")
    iota = lax.iota(jnp.int32, 16)
    zeros = iota * 0

    def in_copy(k, p):
        c0 = wid * PW_COLS + k * WC
        return pltpu.make_async_copy(
            tab_ref.at[:, pl.ds(c0, WC)], slab_v.at[p], in_sems[p])

    def wr_copy(k, p):
        r0 = wid * (PW_COLS // 4) + k * (WC // 4)
        return pltpu.make_async_copy(
            out_v.at[p], rp_ref.at[pl.ds(r0, WC // 4), :], wr_sems[p])

    def transpose(p, nrows):
        def row(rr, c2):
            for s in range(8):
                out_v[p, rr, pl.ds(16 * s, 16)] = plsc.load_gather(
                    slab_v.at[p], [iota + 16 * (s & 1), zeros + (rr * 4 + s // 2)])
            return c2
        lax.fori_loop(0, nrows, row, 0)

    in_copy(0, 0).start()
    in_copy(1, 1).start()

    def chunk(k0, carry):
        for p in range(2):
            k = k0 * 2 + p
            in_copy(k, p).wait()

            @pl.when(k >= 2)
            def _():
                wr_copy(k - 2, p).wait()

            transpose(p, WC // 4)
            wr_copy(k, p).start()

            @pl.when(k + 2 < CHUNKS_PW)
            def _():
                in_copy(k + 2, p).start()
        return carry

    lax.fori_loop(0, (CHUNKS_PW - 1) // 2, chunk, 0)
    in_copy(CHUNKS_PW - 1, 0).wait()
    wr_copy(CHUNKS_PW - 3, 0).wait()
    transpose(0, WC // 4)
    wr_copy(CHUNKS_PW - 1, 0).start()
    wr_copy(CHUNKS_PW - 2, 1).wait()
    wr_copy(CHUNKS_PW - 1, 0).wait()

    @pl.when(wid == 0)
    def _():
        pltpu.sync_copy(tab_ref.at[:, pl.ds(EXTRA_C0, WC)], slab_v.at[0])
        transpose(0, WC // 4)
        pltpu.sync_copy(out_v.at[0],
                        rp_ref.at[pl.ds(EXTRA_C0 // 4, WC // 4), :])

    @pl.when(wid == 1)
    def _():
        pltpu.sync_copy(tail_ref, out_v.at[0, pl.ds(0, TAIL // 4), :])
        pltpu.sync_copy(out_v.at[0, pl.ds(0, TAIL // 4), :],
                        rp_ref.at[pl.ds(TAIL_C0 // 4, TAIL // 4), :])


@functools.partial(
    pl.kernel,
    mesh=_mesh,
    out_type=jax.ShapeDtypeStruct((N_J, EMB_DIM, N_I), jnp.float32),
    scratch_types=[
        pltpu.VMEM((8, 128), jnp.int32),
        pltpu.VMEM((8, 128), jnp.int32),
        pltpu.VMEM((4, 128, 128), jnp.float32),
        pltpu.VMEM((4, EMB_DIM, 128), jnp.float32),
    ] + [pltpu.SemaphoreType.DMA] * 16,
    compiler_params=pltpu.CompilerParams(needs_layout_passes=False),
)
def _sc_gather(tok_ref, rp_ref, out_ref, idx_v, n_v, rows_v, blk_v, *sems):
    idx_sems, gat_sems, out_sems = sems[:8], sems[8:12], sems[12:16]
    wid = lax.axis_index("s") * 2 + lax.axis_index("c")
    iota = lax.iota(jnp.int32, 16)

    def idx_copy(c, s):
        b = wid * BLOCKS_PER_W + c
        return pltpu.make_async_copy(
            tok_ref.at[b // 128, pl.ds((b % 128) * 128, 128)],
            idx_v.at[s], idx_sems[s])

    def gat_copy(c, s, r):
        return pltpu.make_async_copy(
            rp_ref.at[n_v.at[s]], rows_v.at[r], gat_sems[r])

    def out_copy(c, r):
        b = wid * BLOCKS_PER_W + c
        return pltpu.make_async_copy(
            blk_v.at[r], out_ref.at[b // 128, :, pl.ds((b % 128) * 128, 128)],
            out_sems[r])

    def build_n(s):
        def h_body(h, c2):
            v = idx_v[s, pl.ds(h * 16, 16)]
            n_v[s, pl.ds(h * 16, 16)] = lax.shift_right_logical(v, 2)
            return c2
        lax.fori_loop(0, 8, h_body, 0)

    def extract(s, r):
        def h_body(h, c2):
            vo = idx_v[s, pl.ds(h * 16, 16)]
            col = (vo & 3) * 32
            rowi = iota + h * 16
            for d in range(EMB_DIM):
                blk_v[r, d, pl.ds(h * 16, 16)] = plsc.load_gather(
                    rows_v.at[r], [rowi, col + d])
            return c2
        lax.fori_loop(0, 8, h_body, 0)

    for c in range(5):
        idx_copy(c, c % 8).start()
    for c in range(3):
        idx_copy(c, c % 8).wait()
        build_n(c % 8)
        gat_copy(c, c % 8, c % 4).start()

    def outer(c0, carry):
        for u in range(8):
            c = c0 * 8 + u
            s, r = u, u % 4
            gat_copy(c, s, r).wait()

            @pl.when(c >= 4)
            def _():
                out_copy(c - 4, r).wait()

            extract(s, r)
            out_copy(c, r).start()

            @pl.when(c + 5 < BLOCKS_PER_W)
            def _():
                idx_copy(c + 5, (u + 5) % 8).start()

            @pl.when(c + 3 < BLOCKS_PER_W)
            def _():
                idx_copy(c + 3, (u + 3) % 8).wait()
                build_n((u + 3) % 8)
                gat_copy(c + 3, (u + 3) % 8, (u + 3) % 4).start()
        return carry

    lax.fori_loop(0, BLOCKS_PER_W // 8, outer, 0)
    for c in range(BLOCKS_PER_W - 4, BLOCKS_PER_W):
        out_copy(c, c % 4).wait()


def kernel(token_ids, embedding_model):
    tok_t = token_ids.T
    tab_t = embedding_model.T
    tail_rp = lax.slice(embedding_model, (TAIL_C0, 0),
                        (NUM_EMB, EMB_DIM)).reshape(TAIL // 4, 128)
    rp = _sc_repack(tab_t, tail_rp)
    out_t = _sc_gather(tok_t, rp)
    return jnp.transpose(out_t, (2, 0, 1))


# speedup vs baseline: 1.2478x; 1.2478x over previous
import functools

import jax
import jax.numpy as jnp
from jax import lax
from jax.experimental import pallas as pl
from jax.experimental.pallas import tpu as pltpu
from jax.experimental.pallas import tpu_sc as plsc

NUM_EMB = 1_000_000
EMB_DIM = 32
N_I = 16384
N_J = 50
NUM_WORKERS = 32
N_BLOCKS = N_J * (N_I // 128)
BLOCKS_PER_W = N_BLOCKS // NUM_WORKERS

_mesh = plsc.VectorSubcoreMesh(core_axis_name="c", subcore_axis_name="s")


@functools.partial(
    pl.kernel,
    mesh=_mesh,
    out_type=jax.ShapeDtypeStruct((N_J, EMB_DIM, N_I), jnp.float32),
    scratch_types=[
        pltpu.VMEM((8, 128), jnp.int32),
        pltpu.VMEM((8, 128), jnp.int32),
        pltpu.VMEM((4, 128, 128), jnp.float32),
        pltpu.VMEM((4, EMB_DIM, 128), jnp.float32),
    ] + [pltpu.SemaphoreType.DMA] * 16,
    compiler_params=pltpu.CompilerParams(needs_layout_passes=False),
)
def _sc_gather(tok_ref, rp_ref, out_ref, idx_v, n_v, rows_v, blk_v, *sems):
    idx_sems, gat_sems, out_sems = sems[:8], sems[8:12], sems[12:16]
    wid = lax.axis_index("s") * 2 + lax.axis_index("c")
    iota = lax.iota(jnp.int32, 16)

    def idx_copy(c, s):
        b = wid * BLOCKS_PER_W + c
        return pltpu.make_async_copy(
            tok_ref.at[b // 128, pl.ds((b % 128) * 128, 128)],
            idx_v.at[s], idx_sems[s])

    def gat_copy(c, s, r):
        return pltpu.make_async_copy(
            rp_ref.at[n_v.at[s]], rows_v.at[r], gat_sems[r])

    def out_copy(c, r):
        b = wid * BLOCKS_PER_W + c
        return pltpu.make_async_copy(
            blk_v.at[r], out_ref.at[b // 128, :, pl.ds((b % 128) * 128, 128)],
            out_sems[r])

    def build_n(s):
        def h_body(h, c2):
            v = idx_v[s, pl.ds(h * 16, 16)]
            n_v[s, pl.ds(h * 16, 16)] = lax.shift_right_logical(v, 2)
            return c2
        lax.fori_loop(0, 8, h_body, 0)

    def extract(s, r):
        def h_body(h, c2):
            vo = idx_v[s, pl.ds(h * 16, 16)]
            col = (vo & 3) * 32
            rowi = iota + h * 16
            for d in range(EMB_DIM):
                blk_v[r, d, pl.ds(h * 16, 16)] = plsc.load_gather(
                    rows_v.at[r], [rowi, col + d])
            return c2
        lax.fori_loop(0, 8, h_body, 0)

    for c in range(5):
        idx_copy(c, c % 8).start()
    for c in range(3):
        idx_copy(c, c % 8).wait()
        build_n(c % 8)
        gat_copy(c, c % 8, c % 4).start()

    def outer(c0, carry):
        for u in range(8):
            c = c0 * 8 + u
            s, r = u, u % 4
            gat_copy(c, s, r).wait()

            @pl.when(c >= 4)
            def _():
                out_copy(c - 4, r).wait()

            extract(s, r)
            out_copy(c, r).start()

            @pl.when(c + 5 < BLOCKS_PER_W)
            def _():
                idx_copy(c + 5, (u + 5) % 8).start()

            @pl.when(c + 3 < BLOCKS_PER_W)
            def _():
                idx_copy(c + 3, (u + 3) % 8).wait()
                build_n((u + 3) % 8)
                gat_copy(c + 3, (u + 3) % 8, (u + 3) % 4).start()
        return carry

    lax.fori_loop(0, BLOCKS_PER_W // 8, outer, 0)
    for c in range(BLOCKS_PER_W - 4, BLOCKS_PER_W):
        out_copy(c, c % 4).wait()


def kernel(token_ids, embedding_model):
    tok_t = token_ids.T
    rp = embedding_model.reshape(NUM_EMB // 4, 128)
    out_t = _sc_gather(tok_t, rp)
    return jnp.transpose(out_t, (2, 0, 1))
